# Initial kernel scaffold; baseline (speedup 1.0000x reference)
import jax
import jax.numpy as jnp
from jax.experimental import pallas as pl


def kernel(params, snorm_n, label, lb_delta, ub_delta, h, edge_index, e, snorm_e):
    raise NotImplementedError("write your pallas kernel here")


# speedup vs baseline: 3.3853x; 3.3853x over previous
import functools

import jax
import jax.numpy as jnp
from jax import lax
from jax.experimental import pallas as pl
from jax.experimental.pallas import tpu as pltpu
from jax.experimental.pallas import tpu_sc as plsc

N_NODES = 10000
N_PAD = 10240
E = 320000
H = 128
NTILES = 32
CH = 80
E_PAD = NTILES * CH * 128
ROWS_PER_TILE = N_PAD // 16
BLK = 1024
GRID = N_PAD // BLK


_sc_mesh = plsc.VectorSubcoreMesh(core_axis_name="c", subcore_axis_name="s")


@functools.partial(
    pl.kernel,
    mesh=_sc_mesh,
    out_type=jax.ShapeDtypeStruct((2, N_PAD, H), jnp.float32),
    scratch_types=[
        pltpu.VMEM((CH // 2, 128), jnp.int32),
        pltpu.VMEM((CH // 2, 128), jnp.int32),
        pltpu.VMEM((128, H), jnp.float32),
        pltpu.VMEM((128, H), jnp.float32),
        pltpu.VMEM_SHARED((N_PAD, H), jnp.float32),
        pltpu.SemaphoreType.DMA,
        pltpu.SemaphoreType.DMA,
    ],
)
def _segsum(h_hbm, src_hbm, dst_hbm, zeros_hbm, out_hbm,
            src_t, dst_t, rows0, rows1, accum, sem0, sem1):
    c = lax.axis_index("c")
    s = lax.axis_index("s")
    wid = c * 16 + s
    pltpu.sync_copy(zeros_hbm, accum.at[pl.ds(s * ROWS_PER_TILE, ROWS_PER_TILE)])
    plsc.subcore_barrier()

    CHR = CH // 2
    for r in range(2):
        pltpu.sync_copy(src_hbm.at[wid, pl.ds(r * CHR, CHR)], src_t)
        pltpu.sync_copy(dst_hbm.at[wid, pl.ds(r * CHR, CHR)], dst_t)
        pltpu.make_async_copy(h_hbm.at[src_t.at[0]], rows0, sem0).start()

        def body(k, carry):
            j0 = 2 * k
            j1 = j0 + 1
            j2 = j0 + 2
            pltpu.make_async_copy(h_hbm.at[src_t.at[j1]], rows1, sem1).start()
            pltpu.make_async_copy(h_hbm.at[src_t.at[j0]], rows0, sem0).wait()
            pltpu.sync_copy(rows0, accum.at[dst_t.at[j0]], add=True)

            @pl.when(j2 < CHR)
            def _():
                pltpu.make_async_copy(h_hbm.at[src_t.at[j2]], rows0, sem0).start()

            pltpu.make_async_copy(h_hbm.at[src_t.at[j1]], rows1, sem1).wait()
            pltpu.sync_copy(rows1, accum.at[dst_t.at[j1]], add=True)
            return carry

        lax.fori_loop(0, CHR // 2, body, 0)
    plsc.subcore_barrier()
    pltpu.sync_copy(accum.at[pl.ds(s * ROWS_PER_TILE, ROWS_PER_TILE)],
                    out_hbm.at[c, pl.ds(s * ROWS_PER_TILE, ROWS_PER_TILE)])


def _emb_body(ids_ref, emb_ref, out_ref):
    ids = ids_ref[...]
    iota = lax.broadcasted_iota(jnp.int32, (BLK, H), 1)
    onehot = jnp.where(ids == iota, 1.0, 0.0).astype(jnp.float32)
    out_ref[...] = jnp.dot(onehot, emb_ref[...], preferred_element_type=jnp.float32)


def _emb_call(ids, emb):
    return pl.pallas_call(
        _emb_body,
        grid=(GRID,),
        in_specs=[
            pl.BlockSpec((BLK, 1), lambda i: (i, 0)),
            pl.BlockSpec((H, H), lambda i: (0, 0)),
        ],
        out_specs=pl.BlockSpec((BLK, H), lambda i: (i, 0)),
        out_shape=jax.ShapeDtypeStruct((N_PAD, H), jnp.float32),
    )(ids, emb)


def _layer_body(h_ref, nn_ref, sn_ref, eps_ref, w1_ref, c1_ref, w2_ref,
                c2_ref, a3_ref, c3_ref, out_ref):
    h = h_ref[...]
    x = eps_ref[...] * h + nn_ref[0] + nn_ref[1]
    t = jnp.dot(x, w1_ref[...], preferred_element_type=jnp.float32) + c1_ref[...]
    t = jnp.maximum(t, 0.0)
    u = jnp.dot(t, w2_ref[...], preferred_element_type=jnp.float32) + c2_ref[...]
    u = jnp.maximum(u, 0.0)
    y = u * sn_ref[...]
    z = jnp.maximum(y * a3_ref[...] + c3_ref[...], 0.0)
    out_ref[...] = h + z


def _layer_call(h, nn, snorm, eps_row, W1f, c1, W2f, c2, a3, c3):
    row = pl.BlockSpec((1, H), lambda i: (0, 0))
    return pl.pallas_call(
        _layer_body,
        grid=(GRID,),
        in_specs=[
            pl.BlockSpec((BLK, H), lambda i: (i, 0)),
            pl.BlockSpec((2, BLK, H), lambda i: (0, i, 0)),
            pl.BlockSpec((BLK, 1), lambda i: (i, 0)),
            row,
            pl.BlockSpec((H, H), lambda i: (0, 0)),
            row,
            pl.BlockSpec((H, H), lambda i: (0, 0)),
            row, row, row,
        ],
        out_specs=pl.BlockSpec((BLK, H), lambda i: (i, 0)),
        out_shape=jax.ShapeDtypeStruct((N_PAD, H), jnp.float32),
    )(h, nn, snorm, eps_row, W1f, c1, W2f, c2, a3, c3)


def _readout_body(h0_ref, h1_ref, h2_ref, h3_ref, h4_ref, lab_ref,
                  wah_ref, wal_ref, ba_ref, wb_ref, bb_ref, wc_ref, cc_ref,
                  p_ref, bp_ref, lb_ref, ub_ref,
                  sp_ref, gh_ref, w_ref):
    lab = lab_ref[...]
    tlab = jnp.dot(lab, wal_ref[...], preferred_element_type=jnp.float32) + ba_ref[...]
    sp = jnp.zeros((BLK, H), jnp.float32)
    sw = jnp.zeros((BLK, H), jnp.float32)
    for i, href in enumerate((h0_ref, h1_ref, h2_ref, h3_ref, h4_ref)):
        hh = href[...]
        y1 = jnp.maximum(
            jnp.dot(hh, wah_ref[...], preferred_element_type=jnp.float32) + tlab, 0.0)
        y2 = jnp.maximum(
            jnp.dot(y1, wb_ref[...], preferred_element_type=jnp.float32) + bb_ref[...], 0.0)
        sw = sw + jnp.dot(y2, wc_ref[...], preferred_element_type=jnp.float32) + cc_ref[...]
        sp = sp + jnp.dot(hh, p_ref[i], preferred_element_type=jnp.float32) + bp_ref[i][None, :]
    w = jax.nn.sigmoid(sw[:, 0:1])
    lb = lb_ref[...][0:1, 0:1]
    ub = ub_ref[...][0:1, 0:1]
    wt = jnp.minimum(jnp.maximum(w, lb), ub)
    sp_ref[...] = sp
    gh_ref[...] = (1.0 - wt) * lab + wt * 0.1
    w_ref[...] = w


def _readout_call(hs, label_p, WaH, WaL, baP, Wb, bbP, Wc, ccP, Pst, bPst,
                  lbr, ubr):
    blk = pl.BlockSpec((BLK, H), lambda i: (i, 0))
    full = pl.BlockSpec((H, H), lambda i: (0, 0))
    row = pl.BlockSpec((1, H), lambda i: (0, 0))
    return pl.pallas_call(
        _readout_body,
        grid=(GRID,),
        in_specs=[blk, blk, blk, blk, blk,
                  pl.BlockSpec((BLK, 16), lambda i: (i, 0)),
                  full,
                  pl.BlockSpec((16, H), lambda i: (0, 0)),
                  row, full, row, full, row,
                  pl.BlockSpec((5, H, H), lambda i: (0, 0, 0)),
                  pl.BlockSpec((5, H), lambda i: (0, 0)),
                  row, row],
        out_specs=[
            pl.BlockSpec((BLK, H), lambda i: (i, 0)),
            pl.BlockSpec((BLK, 16), lambda i: (i, 0)),
            pl.BlockSpec((BLK, 1), lambda i: (i, 0)),
        ],
        out_shape=[
            jax.ShapeDtypeStruct((N_PAD, H), jnp.float32),
            jax.ShapeDtypeStruct((N_PAD, 16), jnp.float32),
            jax.ShapeDtypeStruct((N_PAD, 1), jnp.float32),
        ],
    )(*hs, label_p, WaH, WaL, baP, Wb, bbP, Wc, ccP, Pst, bPst, lbr, ubr)


def kernel(params, snorm_n, label, lb_delta, ub_delta, h, edge_index, e, snorm_e):
    emb = params['emb']
    ids = jnp.pad(h, (0, N_PAD - N_NODES)).reshape(N_PAD, 1)
    src_p = jnp.concatenate(
        [edge_index[0], jnp.zeros((E_PAD - E,), jnp.int32)]).reshape(NTILES, CH, 128)
    dst_p = jnp.concatenate(
        [edge_index[1], jnp.full((E_PAD - E,), N_NODES, jnp.int32)]).reshape(NTILES, CH, 128)
    zeros_blk = jnp.zeros((ROWS_PER_TILE, H), jnp.float32)
    snorm_p = jnp.pad(snorm_n, ((0, N_PAD - N_NODES), (0, 0)))
    label_p = jnp.pad(label, ((0, N_PAD - N_NODES), (0, 6)))

    h0 = _emb_call(ids, emb)
    hs = [h0]
    s = 1.0 / jnp.sqrt(jnp.float32(1.0 + 1e-5))
    hcur = h0
    for i in range(4):
        p = params['gin'][i]
        a1 = p['mlp_bn_g'] * s
        W1f = p['W1'] * a1[None, :]
        c1 = (p['b1'] * a1 + p['mlp_bn_b'])[None, :]
        a2 = p['apply_bn_g'] * s
        W2f = p['W2'] * a2[None, :]
        c2 = (p['b2'] * a2 + p['apply_bn_b'])[None, :]
        a3 = (p['bn_g'] * s)[None, :]
        c3 = p['bn_b'][None, :]
        eps_row = jnp.full((1, H), 1.0, jnp.float32) + p['eps']
        nn = _segsum(hcur, src_p, dst_p, zeros_blk)
        hcur = _layer_call(hcur, nn, snorm_p, eps_row, W1f, c1, W2f, c2, a3, c3)
        hs.append(hcur)

    Wa = params['w_W'][0]
    WaH = jnp.pad(Wa[:H], ((0, 0), (0, 59)))
    WaL = jnp.pad(Wa[H:], ((0, 6), (0, 59)))
    baP = jnp.pad(params['w_b'][0], (0, 59))[None]
    Wb = jnp.pad(params['w_W'][1], ((0, 59), (0, 94)))
    bbP = jnp.pad(params['w_b'][1], (0, 94))[None]
    Wc = jnp.pad(params['w_W'][2], ((0, 94), (0, 127)))
    ccP = jnp.pad(params['w_b'][2], (0, 127))[None]
    Pst = jnp.stack([jnp.pad(params['pred_W'][i], ((0, 0), (0, 118)))
                     for i in range(5)])
    bPst = jnp.stack([jnp.pad(params['pred_b'][i], (0, 118)) for i in range(5)])
    lbr = jnp.full((1, H), lb_delta, jnp.float32)
    ubr = jnp.full((1, H), ub_delta, jnp.float32)

    sp, ghat, w = _readout_call(hs, label_p, WaH, WaL, baP, Wb, bbP, Wc, ccP,
                                Pst, bPst, lbr, ubr)
    return (sp[:N_NODES, :10], ghat[:N_NODES, :10], edge_index, w[:N_NODES])
